# baseline (speedup 1.0000x reference)
import jax
import jax.numpy as jnp
from jax.experimental import pallas as pl
from jax.experimental.pallas import tpu as pltpu

B, T, K = 512, 512, 128
START, STOP = 126, 127
NEG = -10000.0

B_BLK = 256
NB = B // B_BLK
T_BLK = 64
NT = T // T_BLK
G = 1
BG = B_BLK // G


def _crf_body(feats_ref, tags_ref, tr_ref, trT_ref, logz_ref, gold_ref,
              a_s, logacc_s, ohprev_s, acc_s, ident_s, featT_s):
    it = pl.program_id(1)
    ksub = jax.lax.broadcasted_iota(jnp.int32, (K, B_BLK), 0)

    @pl.when(it == 0)
    def _init():
        a_s[...] = jnp.where(ksub == START, 1.0, 0.0)
        ohprev_s[...] = jnp.where(ksub == START, 1.0, 0.0)
        acc_s[...] = jnp.zeros((K, B_BLK), jnp.float32)
        logacc_s[...] = jnp.zeros((1, B_BLK), jnp.float32)
        bsub = jax.lax.broadcasted_iota(jnp.int32, (B_BLK, B_BLK), 0)
        blane = jax.lax.broadcasted_iota(jnp.int32, (B_BLK, B_BLK), 1)
        ident_s[...] = jnp.where(bsub == blane, 1.0, 0.0)

    tr = tr_ref[...]
    trT = trT_ref[...]
    tmax = jnp.max(jnp.max(tr, axis=1, keepdims=True), axis=0, keepdims=True)
    etc = jnp.exp(tr - tmax)
    ident = ident_s[...]
    ksub_g = ksub[:, :BG]

    def half(jj, slot, a, logacc):
        base = pl.multiple_of(jj * 8, 8)
        fc = feats_ref[:, pl.ds(base, 8), :]
        featT_s[slot] = jax.lax.dot_general(
            fc, ident, (((0,), (0,)), ((), ())),
            preferred_element_type=jnp.float32)

        for s in range(8):
            tagrow = tags_ref[base + s]
            featT = featT_s[slot, s]
            ohT = jnp.where(ksub == tagrow, 1.0, 0.0)
            ef = jnp.exp(featT)

            a_n, la_n = [], []
            for g in range(G):
                lo, hi = g * BG, (g + 1) * BG
                z = jnp.dot(etc, a[g], preferred_element_type=jnp.float32)
                an = z * ef[:, lo:hi]
                la = logacc[g]
                if s % 4 == 3:
                    m4 = jnp.max(an, axis=0, keepdims=True)
                    la = la + jnp.log(m4)
                    an = an * (1.0 / m4)
                a_n.append(an)
                la_n.append(la)
            a, logacc = tuple(a_n), tuple(la_n)

            rows = jnp.dot(trT, ohT, preferred_element_type=jnp.float32)
            acc_s[...] += featT * ohT + rows * ohprev_s[...]
            ohprev_s[...] = ohT
        return a, logacc

    def chunk4(j, carry):
        a, logacc = carry
        for u in range(4):
            a, logacc = half(j * 4 + u, u, a, logacc)
        return a, logacc

    def split(ref):
        v = ref[...]
        return tuple(v[:, g * BG:(g + 1) * BG] for g in range(G))

    carry0 = (split(a_s), split(logacc_s))
    a, logacc = jax.lax.fori_loop(0, T_BLK // 32, chunk4, carry0)
    a_s[...] = jnp.concatenate(a, axis=1)
    logacc_s[...] = jnp.concatenate(logacc, axis=1)

    @pl.when(it == NT - 1)
    def _fin():
        av = a_s[...]
        stop_row = tr_ref[STOP:STOP + 1, :]
        c2 = jnp.max(stop_row, axis=1, keepdims=True)
        estop = jnp.exp(stop_row - c2)
        z = jnp.dot(estop, av, preferred_element_type=jnp.float32)
        logz = logacc_s[...] + c2 + jnp.log(z) + jnp.float32(T) * tmax
        logz_ref[...] = logz.reshape(1, 1, B_BLK)

        stopv = jnp.dot(stop_row, ohprev_s[...],
                        preferred_element_type=jnp.float32)
        gold = jnp.sum(acc_s[...], axis=0, keepdims=True) + stopv
        gold_ref[...] = gold.reshape(1, 1, B_BLK)


def kernel(feats, tags, lengths, transitions):
    del lengths
    tagsT = jnp.transpose(tags.astype(jnp.int32), (1, 0)).reshape(T, 1, B)
    tr = transitions.astype(jnp.float32)
    trT = tr.T

    grid = (NB, NT)
    logz, gold = pl.pallas_call(
        _crf_body,
        grid=grid,
        in_specs=[
            pl.BlockSpec((B_BLK, T_BLK, K), lambda ib, it: (ib, it, 0)),
            pl.BlockSpec((T_BLK, 1, B_BLK), lambda ib, it: (it, 0, ib)),
            pl.BlockSpec((K, K), lambda ib, it: (0, 0)),
            pl.BlockSpec((K, K), lambda ib, it: (0, 0)),
        ],
        out_specs=[
            pl.BlockSpec((1, 1, B_BLK), lambda ib, it: (ib, 0, 0)),
            pl.BlockSpec((1, 1, B_BLK), lambda ib, it: (ib, 0, 0)),
        ],
        out_shape=[
            jax.ShapeDtypeStruct((NB, 1, B_BLK), jnp.float32),
            jax.ShapeDtypeStruct((NB, 1, B_BLK), jnp.float32),
        ],
        scratch_shapes=[pltpu.VMEM((K, B_BLK), jnp.float32),
                        pltpu.VMEM((1, B_BLK), jnp.float32),
                        pltpu.VMEM((K, B_BLK), jnp.float32),
                        pltpu.VMEM((K, B_BLK), jnp.float32),
                        pltpu.VMEM((B_BLK, B_BLK), jnp.float32),
                        pltpu.VMEM((4, 8, K, B_BLK), jnp.float32)],
        compiler_params=pltpu.CompilerParams(
            dimension_semantics=("parallel", "arbitrary"),
        ),
    )(feats, tagsT, tr, trT)
    return jnp.sum(logz) - jnp.sum(gold)


# speedup vs baseline: 1.1212x; 1.1212x over previous
import jax
import jax.numpy as jnp
from jax.experimental import pallas as pl
from jax.experimental.pallas import tpu as pltpu

B, T, K = 512, 512, 128
START, STOP = 126, 127
NEG = -10000.0

B_BLK = 256
NB = B // B_BLK
T_BLK = 64
NT = T // T_BLK
G = 2
BG = B_BLK // G


def _crf_body(feats_ref, tags_ref, tr_ref, trT_ref, logz_ref, gold_ref,
              a_s, logacc_s, ohprev_s, acc_s, ident_s, featT_s):
    it = pl.program_id(1)
    ksub = jax.lax.broadcasted_iota(jnp.int32, (K, B_BLK), 0)

    @pl.when(it == 0)
    def _init():
        a_s[...] = jnp.where(ksub == START, 1.0, 0.0)
        ohprev_s[...] = jnp.where(ksub == START, 1.0, 0.0)
        acc_s[...] = jnp.zeros((K, B_BLK), jnp.float32)
        logacc_s[...] = jnp.zeros((1, B_BLK), jnp.float32)
        bsub = jax.lax.broadcasted_iota(jnp.int32, (B_BLK, B_BLK), 0)
        blane = jax.lax.broadcasted_iota(jnp.int32, (B_BLK, B_BLK), 1)
        ident_s[...] = jnp.where(bsub == blane, 1.0, 0.0)

    tr = tr_ref[...]
    trT = trT_ref[...]
    tmax = jnp.max(jnp.max(tr, axis=1, keepdims=True), axis=0, keepdims=True)
    etc = jnp.exp(tr - tmax)
    ident = ident_s[...]
    ksub_g = ksub[:, :BG]

    def half(jj, slot, a, logacc):
        base = pl.multiple_of(jj * 8, 8)
        fc = feats_ref[:, pl.ds(base, 8), :]
        featT_s[slot] = jax.lax.dot_general(
            fc, ident, (((0,), (0,)), ((), ())),
            preferred_element_type=jnp.float32)

        for s in range(8):
            tagrow = tags_ref[base + s]
            featT = featT_s[slot, s]
            ohT = jnp.where(ksub == tagrow, 1.0, 0.0)
            ef = jnp.exp(featT)

            a_n, la_n = [], []
            for g in range(G):
                lo, hi = g * BG, (g + 1) * BG
                z = jnp.dot(etc, a[g], preferred_element_type=jnp.float32)
                an = z * ef[:, lo:hi]
                la = logacc[g]
                if s % 4 == 3:
                    m4 = jnp.max(an, axis=0, keepdims=True)
                    la = la + jnp.log(m4)
                    an = an * (1.0 / m4)
                a_n.append(an)
                la_n.append(la)
            a, logacc = tuple(a_n), tuple(la_n)

            rows = jnp.dot(trT, ohT, preferred_element_type=jnp.float32)
            acc_s[...] += featT * ohT + rows * ohprev_s[...]
            ohprev_s[...] = ohT
        return a, logacc

    def chunk4(j, carry):
        a, logacc = carry
        for u in range(4):
            a, logacc = half(j * 4 + u, u, a, logacc)
        return a, logacc

    def split(ref):
        v = ref[...]
        return tuple(v[:, g * BG:(g + 1) * BG] for g in range(G))

    carry0 = (split(a_s), split(logacc_s))
    a, logacc = jax.lax.fori_loop(0, T_BLK // 32, chunk4, carry0)
    a_s[...] = jnp.concatenate(a, axis=1)
    logacc_s[...] = jnp.concatenate(logacc, axis=1)

    @pl.when(it == NT - 1)
    def _fin():
        av = a_s[...]
        stop_row = tr_ref[STOP:STOP + 1, :]
        c2 = jnp.max(stop_row, axis=1, keepdims=True)
        estop = jnp.exp(stop_row - c2)
        z = jnp.dot(estop, av, preferred_element_type=jnp.float32)
        logz = logacc_s[...] + c2 + jnp.log(z) + jnp.float32(T) * tmax
        logz_ref[...] = logz.reshape(1, 1, B_BLK)

        stopv = jnp.dot(stop_row, ohprev_s[...],
                        preferred_element_type=jnp.float32)
        gold = jnp.sum(acc_s[...], axis=0, keepdims=True) + stopv
        gold_ref[...] = gold.reshape(1, 1, B_BLK)


def kernel(feats, tags, lengths, transitions):
    del lengths
    tagsT = jnp.transpose(tags.astype(jnp.int32), (1, 0)).reshape(T, 1, B)
    tr = transitions.astype(jnp.float32)
    trT = tr.T

    grid = (NB, NT)
    logz, gold = pl.pallas_call(
        _crf_body,
        grid=grid,
        in_specs=[
            pl.BlockSpec((B_BLK, T_BLK, K), lambda ib, it: (ib, it, 0)),
            pl.BlockSpec((T_BLK, 1, B_BLK), lambda ib, it: (it, 0, ib)),
            pl.BlockSpec((K, K), lambda ib, it: (0, 0)),
            pl.BlockSpec((K, K), lambda ib, it: (0, 0)),
        ],
        out_specs=[
            pl.BlockSpec((1, 1, B_BLK), lambda ib, it: (ib, 0, 0)),
            pl.BlockSpec((1, 1, B_BLK), lambda ib, it: (ib, 0, 0)),
        ],
        out_shape=[
            jax.ShapeDtypeStruct((NB, 1, B_BLK), jnp.float32),
            jax.ShapeDtypeStruct((NB, 1, B_BLK), jnp.float32),
        ],
        scratch_shapes=[pltpu.VMEM((K, B_BLK), jnp.float32),
                        pltpu.VMEM((1, B_BLK), jnp.float32),
                        pltpu.VMEM((K, B_BLK), jnp.float32),
                        pltpu.VMEM((K, B_BLK), jnp.float32),
                        pltpu.VMEM((B_BLK, B_BLK), jnp.float32),
                        pltpu.VMEM((4, 8, K, B_BLK), jnp.float32)],
        compiler_params=pltpu.CompilerParams(
            dimension_semantics=("parallel", "arbitrary"),
            flags={"XLA_TPU_STORE_TO_LOAD_FORWARDING_WINDOW": 12288},
        ),
    )(feats, tagsT, tr, trT)
    return jnp.sum(logz) - jnp.sum(gold)
